# baseline (speedup 1.0000x reference)
import functools
import jax
import jax.numpy as jnp
from jax import lax
from jax.experimental import pallas as pl
from jax.experimental.pallas import tpu as pltpu
from jax.experimental.pallas import tpu_sc as plsc

B, N, F, R, K = 32, 32768, 128, 1024, 128
CH = 8192
NB = N // CH
NV = N // 8
NCH = N // 16


def _scorer_body(h_ref, w_ref, b_ref, out_ref):
    s = jnp.tanh(
        jnp.dot(
            h_ref[...].astype(jnp.bfloat16),
            w_ref[...].astype(jnp.bfloat16),
            preferred_element_type=jnp.float32,
        )
        + b_ref[...][None, :]
    )
    out_ref[...] = s


def _scores_body(emb_ref, sc_ref, out_ref):
    blk = emb_ref[0]
    s = sc_ref[0, 0]
    inv = lax.rsqrt(jnp.sum(s * s))
    s8 = jnp.broadcast_to(s.astype(jnp.bfloat16)[None, :], (8, F))
    r = lax.dot_general(
        s8,
        blk.astype(jnp.bfloat16),
        (((1,), (1,)), ((), ())),
        preferred_element_type=jnp.float32,
    )
    out_ref[0, 0] = r * inv


def _compute_scores(node_embs, h_t, W, b):
    scorer = pl.pallas_call(
        _scorer_body,
        out_shape=jax.ShapeDtypeStruct((B, F), jnp.float32),
    )(h_t, W, b)

    scores = pl.pallas_call(
        _scores_body,
        grid=(B, NB),
        in_specs=[
            pl.BlockSpec((1, CH, F), lambda i, j: (i, j, 0)),
            pl.BlockSpec((1, 1, F), lambda i, j: (i, 0, 0)),
        ],
        out_specs=pl.BlockSpec((1, 1, 8, CH), lambda i, j: (i, j, 0, 0)),
        out_shape=jax.ShapeDtypeStruct((B, NB, 8, CH), jnp.float32),
        compiler_params=pltpu.CompilerParams(
            dimension_semantics=("parallel", "arbitrary")
        ),
    )(node_embs, scorer.reshape(B, 1, F))
    return scores


def _sc_topk(scores_flat, mask_flat, embs_flat):
    mesh = plsc.VectorSubcoreMesh(core_axis_name="c", subcore_axis_name="s")

    @functools.partial(
        pl.kernel,
        mesh=mesh,
        compiler_params=pltpu.CompilerParams(needs_layout_passes=False),
        out_type=[
            jax.ShapeDtypeStruct((B * N,), jnp.float32),
            jax.ShapeDtypeStruct((B * K,), jnp.float32),
            jax.ShapeDtypeStruct((B * K,), jnp.int32),
            jax.ShapeDtypeStruct((B * K, F), jnp.float32),
        ],
        scratch_types=[
            pltpu.VMEM((N,), jnp.float32),
            pltpu.VMEM((N,), jnp.float32),
            pltpu.VMEM((N,), jnp.int32),
            pltpu.VMEM((12288,), jnp.int32),
            pltpu.VMEM((K,), jnp.int32),
            pltpu.VMEM((K,), jnp.float32),
            pltpu.VMEM((K,), jnp.int32),
            pltpu.VMEM((K, F), jnp.float32),
            pltpu.SemaphoreType.DMA,
        ],
    )
    def k(scores_hbm, mask_hbm, embs_hbm, msc_hbm, vals_hbm, idx_hbm,
          rows_hbm, sc_v, mk_v, key_v, hist_v, idx_v, val_v, gidx_v,
          rows_v, sem):
        iota16 = lax.iota(jnp.int32, 16)
        ones16 = jnp.zeros((16,), jnp.int32) + 1
        zeros16 = jnp.zeros((16,), jnp.int32)
        wid = lax.axis_index("s") * 2 + lax.axis_index("c")
        srow0 = wid * (8 * N)
        for j in range(NB):
            pltpu.sync_copy(
                scores_hbm.at[pl.ds(srow0 + j * 8 * CH, CH)],
                sc_v.at[pl.ds(j * CH, CH)],
            )
        pltpu.sync_copy(mask_hbm.at[pl.ds(wid * N, N)], mk_v)

        def zero_hist(nh):
            def zb(i, c):
                hist_v[pl.ds(pl.multiple_of(i * 16, 16), 16)] = zeros16
                return c
            lax.fori_loop(0, 256 * nh, zb, 0)

        zero_hist(3)

        def p0(i, c):
            for u in range(4):
                off = pl.ds(pl.multiple_of((i * 4 + u) * 16, 16), 16)
                sc = sc_v[off] + mk_v[off]
                sc_v[off] = sc
                y = lax.bitcast_convert_type(sc, jnp.int32)
                key = jnp.where(y >= 0, y, y ^ jnp.int32(0x7FFFFFFF))
                key_v[off] = key
                bucket = lax.shift_right_logical(key, 24) ^ 128
                hi = bucket * 16 + iota16 + ((u % 3) * 4096)
                plsc.store_scatter(hist_v, [hi],
                                   plsc.load_gather(hist_v, [hi]) + 1)
            return c

        lax.fori_loop(0, NCH // 4, p0, 0)

        def fold(i, c):
            off0 = pl.ds(pl.multiple_of(i * 16, 16), 16)
            acc = hist_v[off0]
            for u in range(1, 3):
                acc = acc + hist_v[pl.ds(pl.multiple_of(i * 16 + u * 4096, 16), 16)]
            hist_v[off0] = acc
            return c

        lax.fori_loop(0, 256, fold, 0)

        pltpu.sync_copy(sc_v, msc_hbm.at[pl.ds(wid * N, N)])

        def scan_hist(rank_rem):
            def sb(t, carry):
                cum, bstar, above = carry
                bb = 255 - t
                tb = jnp.sum(hist_v[pl.ds(pl.multiple_of(bb * 16, 16), 16)])
                ncum = cum + tb
                hit = (cum < rank_rem) & (ncum >= rank_rem)
                bstar = jnp.where(hit, bb, bstar)
                above = jnp.where(hit, cum, above)
                return (ncum, bstar, above)
            _, bstar, above = lax.fori_loop(0, 256, sb, (0, 0, 0))
            return bstar, above

        b0, above0 = scan_hist(K)
        prefix = lax.shift_left(b0 ^ 128, 24)

        for v in range(K // 16):
            idx_v[pl.ds(v * 16, 16)] = zeros16

        def pc(i, carry):
            c_top, c_cand = carry
            for u in range(4):
                off = pl.ds(pl.multiple_of((i * 4 + u) * 16, 16), 16)
                key = key_v[off]
                bucket = lax.shift_right_logical(key, 24) ^ 128
                ind = (i * 4 + u) * 16 + iota16
                m_top = bucket > b0
                m_cand = bucket == b0
                t32 = m_top.astype(jnp.int32)
                cst = plsc.cumsum(t32)
                pos_t = c_top + cst - 1
                plsc.store_scatter(idx_v, [pos_t], ind,
                                   mask=m_top & (pos_t < K))
                c32 = m_cand.astype(jnp.int32)
                csc = plsc.cumsum(c32)
                pos_c = c_cand + csc - 1
                plsc.store_scatter(sc_v, [pos_c],
                                   lax.bitcast_convert_type(key, jnp.float32),
                                   mask=m_cand)
                plsc.store_scatter(mk_v, [pos_c],
                                   lax.bitcast_convert_type(ind, jnp.float32),
                                   mask=m_cand)
                c_top = c_top + cst[15]
                c_cand = c_cand + csc[15]
            return (c_top, c_cand)

        _, n_cand = lax.fori_loop(0, NCH // 4, pc, (0, 0))

        count_greater = above0

        for p in (1, 2, 3):
            shift = 24 - 8 * p
            himask = jnp.int32(-(1 << (shift + 8)))
            zero_hist(1)
            nit = (n_cand + 15) // 16

            def pr(i, c, himask=himask, shift=shift, prefix=prefix,
                   n_cand=n_cand):
                off = pl.ds(pl.multiple_of(i * 16, 16), 16)
                key = lax.bitcast_convert_type(sc_v[off], jnp.int32)
                valid = (i * 16 + iota16) < n_cand
                cand = ((key & himask) == prefix) & valid
                field = lax.shift_right_logical(key, shift) & 255
                hi = field * 16 + iota16
                plsc.store_scatter(hist_v, [hi],
                                   plsc.load_gather(hist_v, [hi]) + 1,
                                   mask=cand)
                return c

            lax.fori_loop(0, nit, pr, 0)
            bp, abovep = scan_hist(K - count_greater)
            prefix = prefix | lax.shift_left(bp, shift)
            count_greater = count_greater + abovep

        T = prefix

        def pe(i, carry):
            c_gt, c_eq = carry
            off = pl.ds(pl.multiple_of(i * 16, 16), 16)
            key = lax.bitcast_convert_type(sc_v[off], jnp.int32)
            ind = lax.bitcast_convert_type(mk_v[off], jnp.int32)
            valid = (i * 16 + iota16) < n_cand
            m_gt = (key > T) & valid
            m_eq = (key == T) & valid
            g32 = m_gt.astype(jnp.int32)
            pos_g = c_gt + plsc.cumsum(g32) - 1
            plsc.store_scatter(idx_v, [pos_g], ind,
                               mask=m_gt & (pos_g < K))
            e32 = m_eq.astype(jnp.int32)
            pos_e = c_eq + plsc.cumsum(e32) - 1
            plsc.store_scatter(idx_v, [pos_e], ind,
                               mask=m_eq & (pos_e < K))
            return (c_gt + jnp.sum(g32), c_eq + jnp.sum(e32))

        nit2 = (n_cand + 15) // 16
        lax.fori_loop(0, nit2, pe, (above0, count_greater))

        for v in range(K // 16):
            off = pl.ds(v * 16, 16)
            iv = idx_v[off]
            kv = plsc.load_gather(key_v, [iv])
            y = jnp.where(kv >= 0, kv, kv ^ jnp.int32(0x7FFFFFFF))
            val_v[off] = lax.bitcast_convert_type(y, jnp.float32)
            g = iv + wid * N
            gidx_v[off] = jnp.clip(g, 0, B * N - 1)

        pltpu.async_copy(embs_hbm.at[gidx_v], rows_v, sem).wait()

        pltpu.sync_copy(val_v, vals_hbm.at[pl.ds(wid * K, K)])
        pltpu.sync_copy(idx_v, idx_hbm.at[pl.ds(wid * K, K)])
        pltpu.sync_copy(rows_v, rows_hbm.at[pl.ds(wid * K, K)])

    return k(scores_flat, mask_flat, embs_flat)


def _final_body(msc_ref, rows_ref, vals_ref, idx_ref, out_ref, pol_ref):
    i = pl.program_id(0)
    srow = msc_ref[0]
    m = jnp.max(srow)
    lse = m + jnp.log(jnp.sum(jnp.exp(srow - m)))

    v = vals_ref[pl.ds(i, 1), :]
    ix = idx_ref[pl.ds(i, 1), :]
    vc = v[0][:, None]
    ic = ix[0][:, None]
    gt = v > vc
    eq = (v == vc) & (ix < ic)
    rank = jnp.sum((gt | eq).astype(jnp.int32), axis=1, keepdims=True)
    onehot = (rank == lax.broadcasted_iota(jnp.int32, (1, K), 1)).astype(
        jnp.float32
    )
    gate = jnp.tanh(vc)
    s_scaled = rows_ref[0] * gate
    out_ref[0] = lax.dot_general(
        s_scaled,
        onehot,
        (((0,), (0,)), ((), ())),
        preferred_element_type=jnp.float32,
        precision=lax.Precision.HIGHEST,
    )
    pol = jnp.mean(v[0]) - lse
    pol_ref[pl.ds(i, 1), :] = jnp.full((1, 128), pol, jnp.float32)


def kernel(node_embs, mask, h_t, W, b):
    scores = _compute_scores(node_embs, h_t, W, b)
    msc, vals, idxs, rows = _sc_topk(
        scores.reshape(B * NB * 8 * CH),
        mask.reshape(B * N),
        node_embs.reshape(B * N, F),
    )
    out, pol = pl.pallas_call(
        _final_body,
        grid=(B,),
        in_specs=[
            pl.BlockSpec((1, 8, NV), lambda i: (i, 0, 0)),
            pl.BlockSpec((1, K, F), lambda i: (i, 0, 0)),
            pl.BlockSpec((B, K), lambda i: (0, 0)),
            pl.BlockSpec((B, K), lambda i: (0, 0)),
        ],
        out_specs=[
            pl.BlockSpec((1, F, K), lambda i: (i, 0, 0)),
            pl.BlockSpec((B, 128), lambda i: (0, 0)),
        ],
        out_shape=[
            jax.ShapeDtypeStruct((B, F, K), jnp.float32),
            jax.ShapeDtypeStruct((B, 128), jnp.float32),
        ],
    )(
        msc.reshape(B, 8, NV),
        rows.reshape(B, K, F),
        vals.reshape(B, K),
        idxs.reshape(B, K),
    )
    return out, pol[:, 0]


# speedup vs baseline: 1.0149x; 1.0149x over previous
import functools
import jax
import jax.numpy as jnp
from jax import lax
from jax.experimental import pallas as pl
from jax.experimental.pallas import tpu as pltpu
from jax.experimental.pallas import tpu_sc as plsc

B, N, F, R, K = 32, 32768, 128, 1024, 128
CH = 8192
NB = N // CH
NV = N // 8
NCH = N // 16


def _scorer_body(h_ref, w_ref, b_ref, out_ref):
    s = jnp.tanh(
        jnp.dot(
            h_ref[...].astype(jnp.bfloat16),
            w_ref[...].astype(jnp.bfloat16),
            preferred_element_type=jnp.float32,
        )
        + b_ref[...][None, :]
    )
    out_ref[...] = s


def _scores_body(emb_ref, sc_ref, out_ref):
    blk = emb_ref[0]
    s = sc_ref[0, 0]
    inv = lax.rsqrt(jnp.sum(s * s))
    s8 = jnp.broadcast_to(s.astype(jnp.bfloat16)[None, :], (8, F))
    r = lax.dot_general(
        s8,
        blk.astype(jnp.bfloat16),
        (((1,), (1,)), ((), ())),
        preferred_element_type=jnp.float32,
    )
    out_ref[0, 0] = r * inv


def _compute_scores(node_embs, h_t, W, b):
    scorer = pl.pallas_call(
        _scorer_body,
        out_shape=jax.ShapeDtypeStruct((B, F), jnp.float32),
    )(h_t, W, b)

    scores = pl.pallas_call(
        _scores_body,
        grid=(B, NB),
        in_specs=[
            pl.BlockSpec((1, CH, F), lambda i, j: (i, j, 0)),
            pl.BlockSpec((1, 1, F), lambda i, j: (i, 0, 0)),
        ],
        out_specs=pl.BlockSpec((1, 1, 8, CH), lambda i, j: (i, j, 0, 0)),
        out_shape=jax.ShapeDtypeStruct((B, NB, 8, CH), jnp.float32),
        compiler_params=pltpu.CompilerParams(
            dimension_semantics=("parallel", "arbitrary")
        ),
    )(node_embs, scorer.reshape(B, 1, F))
    return scores


def _sc_topk(scores_flat, mask_flat, embs_flat):
    mesh = plsc.VectorSubcoreMesh(core_axis_name="c", subcore_axis_name="s")

    @functools.partial(
        pl.kernel,
        mesh=mesh,
        compiler_params=pltpu.CompilerParams(needs_layout_passes=False),
        out_type=[
            jax.ShapeDtypeStruct((B * N,), jnp.float32),
            jax.ShapeDtypeStruct((B * K,), jnp.float32),
            jax.ShapeDtypeStruct((B * K,), jnp.int32),
            jax.ShapeDtypeStruct((B * K, F), jnp.float32),
        ],
        scratch_types=[
            pltpu.VMEM((N,), jnp.float32),
            pltpu.VMEM((N,), jnp.float32),
            pltpu.VMEM((N,), jnp.int32),
            pltpu.VMEM((4096,), jnp.int32),
            pltpu.VMEM((K,), jnp.int32),
            pltpu.VMEM((K,), jnp.float32),
            pltpu.VMEM((K,), jnp.int32),
            pltpu.VMEM((K, F), jnp.float32),
            pltpu.SemaphoreType.DMA,
        ],
    )
    def k(scores_hbm, mask_hbm, embs_hbm, msc_hbm, vals_hbm, idx_hbm,
          rows_hbm, sc_v, mk_v, key_v, hist_v, idx_v, val_v, gidx_v,
          rows_v, sem):
        iota16 = lax.iota(jnp.int32, 16)
        ones16 = jnp.zeros((16,), jnp.int32) + 1
        zeros16 = jnp.zeros((16,), jnp.int32)
        wid = lax.axis_index("s") * 2 + lax.axis_index("c")
        srow0 = wid * (8 * N)
        for j in range(NB):
            pltpu.sync_copy(
                scores_hbm.at[pl.ds(srow0 + j * 8 * CH, CH)],
                sc_v.at[pl.ds(j * CH, CH)],
            )
        pltpu.sync_copy(mask_hbm.at[pl.ds(wid * N, N)], mk_v)

        def zero_hist():
            def zb(i, c):
                hist_v[pl.ds(pl.multiple_of(i * 16, 16), 16)] = zeros16
                return c
            lax.fori_loop(0, 256, zb, 0)

        zero_hist()

        def p0(i, c):
            off = pl.ds(pl.multiple_of(i * 16, 16), 16)
            sc = sc_v[off] + mk_v[off]
            sc_v[off] = sc
            y = lax.bitcast_convert_type(sc, jnp.int32)
            key = jnp.where(y >= 0, y, y ^ jnp.int32(0x7FFFFFFF))
            key_v[off] = key
            bucket = lax.shift_right_logical(key, 24) ^ 128
            hi = bucket * 16 + iota16
            plsc.store_scatter(hist_v, [hi],
                               plsc.load_gather(hist_v, [hi]) + 1)
            return c

        lax.fori_loop(0, NCH, p0, 0)

        pltpu.sync_copy(sc_v, msc_hbm.at[pl.ds(wid * N, N)])

        def scan_hist(rank_rem):
            def sb(t, carry):
                cum, bstar, above = carry
                bb = 255 - t
                tb = jnp.sum(hist_v[pl.ds(pl.multiple_of(bb * 16, 16), 16)])
                ncum = cum + tb
                hit = (cum < rank_rem) & (ncum >= rank_rem)
                bstar = jnp.where(hit, bb, bstar)
                above = jnp.where(hit, cum, above)
                return (ncum, bstar, above)
            _, bstar, above = lax.fori_loop(0, 256, sb, (0, 0, 0))
            return bstar, above

        b0, above0 = scan_hist(K)
        prefix = lax.shift_left(b0 ^ 128, 24)

        for v in range(K // 16):
            idx_v[pl.ds(v * 16, 16)] = zeros16

        def pc(i, carry):
            c_top, c_cand = carry
            off = pl.ds(pl.multiple_of(i * 16, 16), 16)
            key = key_v[off]
            bucket = lax.shift_right_logical(key, 24) ^ 128
            ind = i * 16 + iota16
            m_top = bucket > b0
            m_cand = bucket == b0
            t32 = m_top.astype(jnp.int32)
            pos_t = c_top + plsc.cumsum(t32) - 1
            plsc.store_scatter(idx_v, [pos_t], ind,
                               mask=m_top & (pos_t < K))
            c32 = m_cand.astype(jnp.int32)
            pos_c = c_cand + plsc.cumsum(c32) - 1
            plsc.store_scatter(sc_v, [pos_c],
                               lax.bitcast_convert_type(key, jnp.float32), mask=m_cand)
            plsc.store_scatter(mk_v, [pos_c],
                               lax.bitcast_convert_type(ind, jnp.float32), mask=m_cand)
            return (c_top + jnp.sum(t32), c_cand + jnp.sum(c32))

        _, n_cand = lax.fori_loop(0, NCH, pc, (0, 0))

        count_greater = above0

        for p in (1, 2, 3):
            shift = 24 - 8 * p
            himask = jnp.int32(-(1 << (shift + 8)))
            zero_hist()
            nit = (n_cand + 15) // 16

            def pr(i, c, himask=himask, shift=shift, prefix=prefix,
                   n_cand=n_cand):
                off = pl.ds(pl.multiple_of(i * 16, 16), 16)
                key = lax.bitcast_convert_type(sc_v[off], jnp.int32)
                valid = (i * 16 + iota16) < n_cand
                cand = ((key & himask) == prefix) & valid
                field = lax.shift_right_logical(key, shift) & 255
                hi = field * 16 + iota16
                plsc.store_scatter(hist_v, [hi],
                                   plsc.load_gather(hist_v, [hi]) + 1,
                                   mask=cand)
                return c

            lax.fori_loop(0, nit, pr, 0)
            bp, abovep = scan_hist(K - count_greater)
            prefix = prefix | lax.shift_left(bp, shift)
            count_greater = count_greater + abovep

        T = prefix

        def pe(i, carry):
            c_gt, c_eq = carry
            off = pl.ds(pl.multiple_of(i * 16, 16), 16)
            key = lax.bitcast_convert_type(sc_v[off], jnp.int32)
            ind = lax.bitcast_convert_type(mk_v[off], jnp.int32)
            valid = (i * 16 + iota16) < n_cand
            m_gt = (key > T) & valid
            m_eq = (key == T) & valid
            g32 = m_gt.astype(jnp.int32)
            pos_g = c_gt + plsc.cumsum(g32) - 1
            plsc.store_scatter(idx_v, [pos_g], ind,
                               mask=m_gt & (pos_g < K))
            e32 = m_eq.astype(jnp.int32)
            pos_e = c_eq + plsc.cumsum(e32) - 1
            plsc.store_scatter(idx_v, [pos_e], ind,
                               mask=m_eq & (pos_e < K))
            return (c_gt + jnp.sum(g32), c_eq + jnp.sum(e32))

        nit2 = (n_cand + 15) // 16
        lax.fori_loop(0, nit2, pe, (above0, count_greater))

        for v in range(K // 16):
            off = pl.ds(v * 16, 16)
            iv = idx_v[off]
            kv = plsc.load_gather(key_v, [iv])
            y = jnp.where(kv >= 0, kv, kv ^ jnp.int32(0x7FFFFFFF))
            val_v[off] = lax.bitcast_convert_type(y, jnp.float32)
            g = iv + wid * N
            gidx_v[off] = jnp.clip(g, 0, B * N - 1)

        pltpu.async_copy(embs_hbm.at[gidx_v], rows_v, sem).wait()

        pltpu.sync_copy(val_v, vals_hbm.at[pl.ds(wid * K, K)])
        pltpu.sync_copy(idx_v, idx_hbm.at[pl.ds(wid * K, K)])
        pltpu.sync_copy(rows_v, rows_hbm.at[pl.ds(wid * K, K)])

    return k(scores_flat, mask_flat, embs_flat)


def _final_body(msc_ref, rows_ref, vals_ref, idx_ref, out_ref, pol_ref):
    i = pl.program_id(0)
    srow = msc_ref[0]
    m = jnp.max(srow)
    lse = m + jnp.log(jnp.sum(jnp.exp(srow - m)))

    v = vals_ref[pl.ds(i, 1), :]
    ix = idx_ref[pl.ds(i, 1), :]
    vc = v[0][:, None]
    ic = ix[0][:, None]
    gt = v > vc
    eq = (v == vc) & (ix < ic)
    rank = jnp.sum((gt | eq).astype(jnp.int32), axis=1, keepdims=True)
    onehot = (rank == lax.broadcasted_iota(jnp.int32, (1, K), 1)).astype(
        jnp.float32
    )
    gate = jnp.tanh(vc)
    s_scaled = rows_ref[0] * gate
    out_ref[0] = lax.dot_general(
        s_scaled,
        onehot,
        (((0,), (0,)), ((), ())),
        preferred_element_type=jnp.float32,
        precision=lax.Precision.HIGHEST,
    )
    pol = jnp.mean(v[0]) - lse
    pol_ref[pl.ds(i, 1), :] = jnp.full((1, 128), pol, jnp.float32)


def kernel(node_embs, mask, h_t, W, b):
    scores = _compute_scores(node_embs, h_t, W, b)
    msc, vals, idxs, rows = _sc_topk(
        scores.reshape(B * NB * 8 * CH),
        mask.reshape(B * N),
        node_embs.reshape(B * N, F),
    )
    out, pol = pl.pallas_call(
        _final_body,
        grid=(B,),
        in_specs=[
            pl.BlockSpec((1, 8, NV), lambda i: (i, 0, 0)),
            pl.BlockSpec((1, K, F), lambda i: (i, 0, 0)),
            pl.BlockSpec((B, K), lambda i: (0, 0)),
            pl.BlockSpec((B, K), lambda i: (0, 0)),
        ],
        out_specs=[
            pl.BlockSpec((1, F, K), lambda i: (i, 0, 0)),
            pl.BlockSpec((B, 128), lambda i: (0, 0)),
        ],
        out_shape=[
            jax.ShapeDtypeStruct((B, F, K), jnp.float32),
            jax.ShapeDtypeStruct((B, 128), jnp.float32),
        ],
    )(
        msc.reshape(B, 8, NV),
        rows.reshape(B, K, F),
        vals.reshape(B, K),
        idxs.reshape(B, K),
    )
    return out, pol[:, 0]


# speedup vs baseline: 1.1340x; 1.1173x over previous
import functools
import jax
import jax.numpy as jnp
from jax import lax
from jax.experimental import pallas as pl
from jax.experimental.pallas import tpu as pltpu
from jax.experimental.pallas import tpu_sc as plsc

B, N, F, R, K = 32, 32768, 128, 1024, 128
CH = 8192
NB = N // CH
NV = N // 8
NCH = N // 16


def _scorer_body(h_ref, w_ref, b_ref, out_ref):
    s = jnp.tanh(
        jnp.dot(
            h_ref[...].astype(jnp.bfloat16),
            w_ref[...].astype(jnp.bfloat16),
            preferred_element_type=jnp.float32,
        )
        + b_ref[...][None, :]
    )
    out_ref[...] = s


def _scores_body(emb_ref, sc_ref, out_ref):
    blk = emb_ref[0]
    s = sc_ref[0, 0]
    inv = lax.rsqrt(jnp.sum(s * s))
    s8 = jnp.broadcast_to(s.astype(jnp.bfloat16)[None, :], (8, F))
    r = lax.dot_general(
        s8,
        blk.astype(jnp.bfloat16),
        (((1,), (1,)), ((), ())),
        preferred_element_type=jnp.float32,
    )
    out_ref[0] = jnp.reshape(r[0], (CH // 128, 128)) * inv


def _compute_scores(node_embs, h_t, W, b):
    scorer = pl.pallas_call(
        _scorer_body,
        out_shape=jax.ShapeDtypeStruct((B, F), jnp.float32),
    )(h_t, W, b)

    scores = pl.pallas_call(
        _scores_body,
        grid=(B, NB),
        in_specs=[
            pl.BlockSpec((1, CH, F), lambda i, j: (i, j, 0)),
            pl.BlockSpec((1, 1, F), lambda i, j: (i, 0, 0)),
        ],
        out_specs=pl.BlockSpec((1, CH // 128, 128), lambda i, j: (i, j, 0)),
        out_shape=jax.ShapeDtypeStruct((B, N // 128, 128), jnp.float32),
        compiler_params=pltpu.CompilerParams(
            dimension_semantics=("parallel", "arbitrary")
        ),
    )(node_embs, scorer.reshape(B, 1, F))
    return scores


def _sc_topk(scores_flat, mask_flat, embs_flat):
    mesh = plsc.VectorSubcoreMesh(core_axis_name="c", subcore_axis_name="s")

    @functools.partial(
        pl.kernel,
        mesh=mesh,
        compiler_params=pltpu.CompilerParams(needs_layout_passes=False),
        out_type=[
            jax.ShapeDtypeStruct((B * N,), jnp.float32),
            jax.ShapeDtypeStruct((B * K,), jnp.float32),
            jax.ShapeDtypeStruct((B * K,), jnp.int32),
            jax.ShapeDtypeStruct((B * K, F), jnp.float32),
        ],
        scratch_types=[
            pltpu.VMEM((N,), jnp.float32),
            pltpu.VMEM((N,), jnp.float32),
            pltpu.VMEM((N,), jnp.int32),
            pltpu.VMEM((4096,), jnp.int32),
            pltpu.VMEM((K,), jnp.int32),
            pltpu.VMEM((K,), jnp.float32),
            pltpu.VMEM((K,), jnp.int32),
            pltpu.VMEM((K, F), jnp.float32),
            pltpu.SemaphoreType.DMA,
        ],
    )
    def k(scores_hbm, mask_hbm, embs_hbm, msc_hbm, vals_hbm, idx_hbm,
          rows_hbm, sc_v, mk_v, key_v, hist_v, idx_v, val_v, gidx_v,
          rows_v, sem):
        iota16 = lax.iota(jnp.int32, 16)
        ones16 = jnp.zeros((16,), jnp.int32) + 1
        zeros16 = jnp.zeros((16,), jnp.int32)
        wid = lax.axis_index("s") * 2 + lax.axis_index("c")
        pltpu.sync_copy(scores_hbm.at[pl.ds(wid * N, N)], sc_v)
        pltpu.sync_copy(mask_hbm.at[pl.ds(wid * N, N)], mk_v)

        def zero_hist():
            def zb(i, c):
                hist_v[pl.ds(pl.multiple_of(i * 16, 16), 16)] = zeros16
                return c
            lax.fori_loop(0, 256, zb, 0)

        zero_hist()

        def p0(i, c):
            off = pl.ds(pl.multiple_of(i * 16, 16), 16)
            sc = sc_v[off] + mk_v[off]
            sc_v[off] = sc
            y = lax.bitcast_convert_type(sc, jnp.int32)
            key = jnp.where(y >= 0, y, y ^ jnp.int32(0x7FFFFFFF))
            key_v[off] = key
            bucket = lax.shift_right_logical(key, 24) ^ 128
            hi = bucket * 16 + iota16
            plsc.store_scatter(hist_v, [hi],
                               plsc.load_gather(hist_v, [hi]) + 1)
            return c

        lax.fori_loop(0, NCH, p0, 0)

        pltpu.sync_copy(sc_v, msc_hbm.at[pl.ds(wid * N, N)])

        def scan_hist(rank_rem):
            def sb(t, carry):
                cum, bstar, above = carry
                bb = 255 - t
                tb = jnp.sum(hist_v[pl.ds(pl.multiple_of(bb * 16, 16), 16)])
                ncum = cum + tb
                hit = (cum < rank_rem) & (ncum >= rank_rem)
                bstar = jnp.where(hit, bb, bstar)
                above = jnp.where(hit, cum, above)
                return (ncum, bstar, above)
            _, bstar, above = lax.fori_loop(0, 256, sb, (0, 0, 0))
            return bstar, above

        b0, above0 = scan_hist(K)
        prefix = lax.shift_left(b0 ^ 128, 24)

        for v in range(K // 16):
            idx_v[pl.ds(v * 16, 16)] = zeros16

        def pc(i, carry):
            c_top, c_cand = carry
            off = pl.ds(pl.multiple_of(i * 16, 16), 16)
            key = key_v[off]
            bucket = lax.shift_right_logical(key, 24) ^ 128
            ind = i * 16 + iota16
            m_top = bucket > b0
            m_cand = bucket == b0
            t32 = m_top.astype(jnp.int32)
            pos_t = c_top + plsc.cumsum(t32) - 1
            plsc.store_scatter(idx_v, [pos_t], ind,
                               mask=m_top & (pos_t < K))
            c32 = m_cand.astype(jnp.int32)
            pos_c = c_cand + plsc.cumsum(c32) - 1
            plsc.store_scatter(sc_v, [pos_c],
                               lax.bitcast_convert_type(key, jnp.float32), mask=m_cand)
            plsc.store_scatter(mk_v, [pos_c],
                               lax.bitcast_convert_type(ind, jnp.float32), mask=m_cand)
            return (c_top + jnp.sum(t32), c_cand + jnp.sum(c32))

        _, n_cand = lax.fori_loop(0, NCH, pc, (0, 0))

        count_greater = above0

        for p in (1, 2, 3):
            shift = 24 - 8 * p
            himask = jnp.int32(-(1 << (shift + 8)))
            zero_hist()
            nit = (n_cand + 15) // 16

            def pr(i, c, himask=himask, shift=shift, prefix=prefix,
                   n_cand=n_cand):
                off = pl.ds(pl.multiple_of(i * 16, 16), 16)
                key = lax.bitcast_convert_type(sc_v[off], jnp.int32)
                valid = (i * 16 + iota16) < n_cand
                cand = ((key & himask) == prefix) & valid
                field = lax.shift_right_logical(key, shift) & 255
                hi = field * 16 + iota16
                plsc.store_scatter(hist_v, [hi],
                                   plsc.load_gather(hist_v, [hi]) + 1,
                                   mask=cand)
                return c

            lax.fori_loop(0, nit, pr, 0)
            bp, abovep = scan_hist(K - count_greater)
            prefix = prefix | lax.shift_left(bp, shift)
            count_greater = count_greater + abovep

        T = prefix

        def pe(i, carry):
            c_gt, c_eq = carry
            off = pl.ds(pl.multiple_of(i * 16, 16), 16)
            key = lax.bitcast_convert_type(sc_v[off], jnp.int32)
            ind = lax.bitcast_convert_type(mk_v[off], jnp.int32)
            valid = (i * 16 + iota16) < n_cand
            m_gt = (key > T) & valid
            m_eq = (key == T) & valid
            g32 = m_gt.astype(jnp.int32)
            pos_g = c_gt + plsc.cumsum(g32) - 1
            plsc.store_scatter(idx_v, [pos_g], ind,
                               mask=m_gt & (pos_g < K))
            e32 = m_eq.astype(jnp.int32)
            pos_e = c_eq + plsc.cumsum(e32) - 1
            plsc.store_scatter(idx_v, [pos_e], ind,
                               mask=m_eq & (pos_e < K))
            return (c_gt + jnp.sum(g32), c_eq + jnp.sum(e32))

        nit2 = (n_cand + 15) // 16
        lax.fori_loop(0, nit2, pe, (above0, count_greater))

        for v in range(K // 16):
            off = pl.ds(v * 16, 16)
            iv = idx_v[off]
            kv = plsc.load_gather(key_v, [iv])
            y = jnp.where(kv >= 0, kv, kv ^ jnp.int32(0x7FFFFFFF))
            val_v[off] = lax.bitcast_convert_type(y, jnp.float32)
            g = iv + wid * N
            gidx_v[off] = jnp.clip(g, 0, B * N - 1)

        pltpu.async_copy(embs_hbm.at[gidx_v], rows_v, sem).wait()

        pltpu.sync_copy(val_v, vals_hbm.at[pl.ds(wid * K, K)])
        pltpu.sync_copy(idx_v, idx_hbm.at[pl.ds(wid * K, K)])
        pltpu.sync_copy(rows_v, rows_hbm.at[pl.ds(wid * K, K)])

    return k(scores_flat, mask_flat, embs_flat)


def _final_body(msc_ref, rows_ref, vals_ref, idx_ref, out_ref, pol_ref):
    i = pl.program_id(0)
    srow = msc_ref[0]
    m = jnp.max(srow)
    lse = m + jnp.log(jnp.sum(jnp.exp(srow - m)))

    v = vals_ref[pl.ds(i, 1), :]
    ix = idx_ref[pl.ds(i, 1), :]
    vc = v[0][:, None]
    ic = ix[0][:, None]
    gt = v > vc
    eq = (v == vc) & (ix < ic)
    rank = jnp.sum((gt | eq).astype(jnp.int32), axis=1, keepdims=True)
    onehot = (rank == lax.broadcasted_iota(jnp.int32, (1, K), 1)).astype(
        jnp.float32
    )
    gate = jnp.tanh(vc)
    s_scaled = rows_ref[0] * gate
    out_ref[0] = lax.dot_general(
        s_scaled,
        onehot,
        (((0,), (0,)), ((), ())),
        preferred_element_type=jnp.float32,
        precision=lax.Precision.HIGHEST,
    )
    pol = jnp.mean(v[0]) - lse
    pol_ref[pl.ds(i, 1), :] = jnp.full((1, 128), pol, jnp.float32)


def kernel(node_embs, mask, h_t, W, b):
    scores = _compute_scores(node_embs, h_t, W, b)
    msc, vals, idxs, rows = _sc_topk(
        scores.reshape(B * N),
        mask.reshape(B * N),
        node_embs.reshape(B * N, F),
    )
    out, pol = pl.pallas_call(
        _final_body,
        grid=(B,),
        in_specs=[
            pl.BlockSpec((1, 8, NV), lambda i: (i, 0, 0)),
            pl.BlockSpec((1, K, F), lambda i: (i, 0, 0)),
            pl.BlockSpec((B, K), lambda i: (0, 0)),
            pl.BlockSpec((B, K), lambda i: (0, 0)),
        ],
        out_specs=[
            pl.BlockSpec((1, F, K), lambda i: (i, 0, 0)),
            pl.BlockSpec((B, 128), lambda i: (0, 0)),
        ],
        out_shape=[
            jax.ShapeDtypeStruct((B, F, K), jnp.float32),
            jax.ShapeDtypeStruct((B, 128), jnp.float32),
        ],
    )(
        msc.reshape(B, 8, NV),
        rows.reshape(B, K, F),
        vals.reshape(B, K),
        idxs.reshape(B, K),
    )
    return out, pol[:, 0]


# speedup vs baseline: 1.1494x; 1.0136x over previous
import functools
import jax
import jax.numpy as jnp
from jax import lax
from jax.experimental import pallas as pl
from jax.experimental.pallas import tpu as pltpu
from jax.experimental.pallas import tpu_sc as plsc

B, N, F, R, K = 32, 32768, 128, 1024, 128
CH = 8192
NB = N // CH
NV = N // 8
NCH = N // 16


def _scorer_body(h_ref, w_ref, b_ref, out_ref):
    s = jnp.tanh(
        jnp.dot(
            h_ref[...].astype(jnp.bfloat16),
            w_ref[...].astype(jnp.bfloat16),
            preferred_element_type=jnp.float32,
        )
        + b_ref[...][None, :]
    )
    out_ref[...] = s


def _scores_body(emb_ref, sc_ref, out_ref):
    blk = emb_ref[0]
    s = sc_ref[0, 0]
    inv = lax.rsqrt(jnp.sum(s * s))
    s8 = jnp.broadcast_to(s.astype(jnp.bfloat16)[None, :], (8, F))
    r = lax.dot_general(
        s8,
        blk.astype(jnp.bfloat16),
        (((1,), (1,)), ((), ())),
        preferred_element_type=jnp.float32,
    )
    out_ref[0] = jnp.reshape(r[0], (CH // 128, 128)) * inv


def _compute_scores(node_embs, h_t, W, b):
    scorer = pl.pallas_call(
        _scorer_body,
        out_shape=jax.ShapeDtypeStruct((B, F), jnp.float32),
    )(h_t, W, b)

    scores = pl.pallas_call(
        _scores_body,
        grid=(B, NB),
        in_specs=[
            pl.BlockSpec((1, CH, F), lambda i, j: (i, j, 0)),
            pl.BlockSpec((1, 1, F), lambda i, j: (i, 0, 0)),
        ],
        out_specs=pl.BlockSpec((1, CH // 128, 128), lambda i, j: (i, j, 0)),
        out_shape=jax.ShapeDtypeStruct((B, N // 128, 128), jnp.float32),
        compiler_params=pltpu.CompilerParams(
            dimension_semantics=("parallel", "arbitrary")
        ),
    )(node_embs, scorer.reshape(B, 1, F))
    return scores


def _sc_topk(scores_flat, mask_flat, embs_flat):
    mesh = plsc.VectorSubcoreMesh(core_axis_name="c", subcore_axis_name="s")

    @functools.partial(
        pl.kernel,
        mesh=mesh,
        compiler_params=pltpu.CompilerParams(needs_layout_passes=False),
        out_type=[
            jax.ShapeDtypeStruct((B * N,), jnp.float32),
            jax.ShapeDtypeStruct((B * K,), jnp.float32),
            jax.ShapeDtypeStruct((B * K,), jnp.int32),
            jax.ShapeDtypeStruct((B * K, F), jnp.float32),
        ],
        scratch_types=[
            pltpu.VMEM((N,), jnp.float32),
            pltpu.VMEM((N,), jnp.float32),
            pltpu.VMEM((N,), jnp.int32),
            pltpu.VMEM((4096,), jnp.int32),
            pltpu.VMEM((K,), jnp.int32),
            pltpu.VMEM((K,), jnp.float32),
            pltpu.VMEM((K,), jnp.int32),
            pltpu.VMEM((K, F), jnp.float32),
            pltpu.SemaphoreType.DMA,
        ],
    )
    def k(scores_hbm, mask_hbm, embs_hbm, msc_hbm, vals_hbm, idx_hbm,
          rows_hbm, sc_v, mk_v, key_v, hist_v, idx_v, val_v, gidx_v,
          rows_v, sem):
        iota16 = lax.iota(jnp.int32, 16)
        ones16 = jnp.zeros((16,), jnp.int32) + 1
        zeros16 = jnp.zeros((16,), jnp.int32)
        wid = lax.axis_index("s") * 2 + lax.axis_index("c")
        pltpu.sync_copy(scores_hbm.at[pl.ds(wid * N, N)], sc_v)
        pltpu.sync_copy(mask_hbm.at[pl.ds(wid * N, N)], mk_v)

        def zero_hist():
            def zb(i, c):
                hist_v[pl.ds(pl.multiple_of(i * 16, 16), 16)] = zeros16
                return c
            lax.fori_loop(0, 256, zb, 0)

        zero_hist()

        def p0(i, c):
            off = pl.ds(pl.multiple_of(i * 16, 16), 16)
            sc = sc_v[off] + mk_v[off]
            sc_v[off] = sc
            y = lax.bitcast_convert_type(sc, jnp.int32)
            key = jnp.where(y >= 0, y, y ^ jnp.int32(0x7FFFFFFF))
            key_v[off] = key
            bucket = lax.shift_right_logical(key, 24) ^ 128
            plsc.addupdate_scatter(hist_v, [bucket * 16 + iota16], ones16)
            return c

        lax.fori_loop(0, NCH, p0, 0)

        pltpu.sync_copy(sc_v, msc_hbm.at[pl.ds(wid * N, N)])

        def scan_hist(rank_rem):
            def sb(t, carry):
                cum, bstar, above = carry
                bb = 255 - t
                tb = jnp.sum(hist_v[pl.ds(pl.multiple_of(bb * 16, 16), 16)])
                ncum = cum + tb
                hit = (cum < rank_rem) & (ncum >= rank_rem)
                bstar = jnp.where(hit, bb, bstar)
                above = jnp.where(hit, cum, above)
                return (ncum, bstar, above)
            _, bstar, above = lax.fori_loop(0, 256, sb, (0, 0, 0))
            return bstar, above

        b0, above0 = scan_hist(K)
        prefix = lax.shift_left(b0 ^ 128, 24)

        for v in range(K // 16):
            idx_v[pl.ds(v * 16, 16)] = zeros16

        def pc(i, carry):
            c_top, c_cand = carry
            off = pl.ds(pl.multiple_of(i * 16, 16), 16)
            key = key_v[off]
            bucket = lax.shift_right_logical(key, 24) ^ 128
            ind = i * 16 + iota16
            m_top = bucket > b0
            m_cand = bucket == b0
            t32 = m_top.astype(jnp.int32)
            pos_t = c_top + plsc.cumsum(t32) - 1
            plsc.store_scatter(idx_v, [pos_t], ind,
                               mask=m_top & (pos_t < K))
            c32 = m_cand.astype(jnp.int32)
            pos_c = c_cand + plsc.cumsum(c32) - 1
            plsc.store_scatter(sc_v, [pos_c],
                               lax.bitcast_convert_type(key, jnp.float32), mask=m_cand)
            plsc.store_scatter(mk_v, [pos_c],
                               lax.bitcast_convert_type(ind, jnp.float32), mask=m_cand)
            return (c_top + jnp.sum(t32), c_cand + jnp.sum(c32))

        _, n_cand = lax.fori_loop(0, NCH, pc, (0, 0))

        count_greater = above0

        for p in (1, 2, 3):
            shift = 24 - 8 * p
            himask = jnp.int32(-(1 << (shift + 8)))
            zero_hist()
            nit = (n_cand + 15) // 16

            def pr(i, c, himask=himask, shift=shift, prefix=prefix,
                   n_cand=n_cand):
                off = pl.ds(pl.multiple_of(i * 16, 16), 16)
                key = lax.bitcast_convert_type(sc_v[off], jnp.int32)
                valid = (i * 16 + iota16) < n_cand
                cand = ((key & himask) == prefix) & valid
                field = lax.shift_right_logical(key, shift) & 255
                plsc.addupdate_scatter(hist_v, [field * 16 + iota16],
                                       ones16, mask=cand)
                return c

            lax.fori_loop(0, nit, pr, 0)
            bp, abovep = scan_hist(K - count_greater)
            prefix = prefix | lax.shift_left(bp, shift)
            count_greater = count_greater + abovep

        T = prefix

        def pe(i, carry):
            c_gt, c_eq = carry
            off = pl.ds(pl.multiple_of(i * 16, 16), 16)
            key = lax.bitcast_convert_type(sc_v[off], jnp.int32)
            ind = lax.bitcast_convert_type(mk_v[off], jnp.int32)
            valid = (i * 16 + iota16) < n_cand
            m_gt = (key > T) & valid
            m_eq = (key == T) & valid
            g32 = m_gt.astype(jnp.int32)
            pos_g = c_gt + plsc.cumsum(g32) - 1
            plsc.store_scatter(idx_v, [pos_g], ind,
                               mask=m_gt & (pos_g < K))
            e32 = m_eq.astype(jnp.int32)
            pos_e = c_eq + plsc.cumsum(e32) - 1
            plsc.store_scatter(idx_v, [pos_e], ind,
                               mask=m_eq & (pos_e < K))
            return (c_gt + jnp.sum(g32), c_eq + jnp.sum(e32))

        nit2 = (n_cand + 15) // 16
        lax.fori_loop(0, nit2, pe, (above0, count_greater))

        for v in range(K // 16):
            off = pl.ds(v * 16, 16)
            iv = idx_v[off]
            kv = plsc.load_gather(key_v, [iv])
            y = jnp.where(kv >= 0, kv, kv ^ jnp.int32(0x7FFFFFFF))
            val_v[off] = lax.bitcast_convert_type(y, jnp.float32)
            g = iv + wid * N
            gidx_v[off] = jnp.clip(g, 0, B * N - 1)

        pltpu.async_copy(embs_hbm.at[gidx_v], rows_v, sem).wait()

        pltpu.sync_copy(val_v, vals_hbm.at[pl.ds(wid * K, K)])
        pltpu.sync_copy(idx_v, idx_hbm.at[pl.ds(wid * K, K)])
        pltpu.sync_copy(rows_v, rows_hbm.at[pl.ds(wid * K, K)])

    return k(scores_flat, mask_flat, embs_flat)


def _final_body(msc_ref, rows_ref, vals_ref, idx_ref, out_ref, pol_ref):
    i = pl.program_id(0)
    srow = msc_ref[0]
    m = jnp.max(srow)
    lse = m + jnp.log(jnp.sum(jnp.exp(srow - m)))

    v = vals_ref[pl.ds(i, 1), :]
    ix = idx_ref[pl.ds(i, 1), :]
    vc = v[0][:, None]
    ic = ix[0][:, None]
    gt = v > vc
    eq = (v == vc) & (ix < ic)
    rank = jnp.sum((gt | eq).astype(jnp.int32), axis=1, keepdims=True)
    onehot = (rank == lax.broadcasted_iota(jnp.int32, (1, K), 1)).astype(
        jnp.float32
    )
    gate = jnp.tanh(vc)
    s_scaled = rows_ref[0] * gate
    out_ref[0] = lax.dot_general(
        s_scaled,
        onehot,
        (((0,), (0,)), ((), ())),
        preferred_element_type=jnp.float32,
        precision=lax.Precision.HIGHEST,
    )
    pol = jnp.mean(v[0]) - lse
    pol_ref[pl.ds(i, 1), :] = jnp.full((1, 128), pol, jnp.float32)


def kernel(node_embs, mask, h_t, W, b):
    scores = _compute_scores(node_embs, h_t, W, b)
    msc, vals, idxs, rows = _sc_topk(
        scores.reshape(B * N),
        mask.reshape(B * N),
        node_embs.reshape(B * N, F),
    )
    out, pol = pl.pallas_call(
        _final_body,
        grid=(B,),
        in_specs=[
            pl.BlockSpec((1, 8, NV), lambda i: (i, 0, 0)),
            pl.BlockSpec((1, K, F), lambda i: (i, 0, 0)),
            pl.BlockSpec((B, K), lambda i: (0, 0)),
            pl.BlockSpec((B, K), lambda i: (0, 0)),
        ],
        out_specs=[
            pl.BlockSpec((1, F, K), lambda i: (i, 0, 0)),
            pl.BlockSpec((B, 128), lambda i: (0, 0)),
        ],
        out_shape=[
            jax.ShapeDtypeStruct((B, F, K), jnp.float32),
            jax.ShapeDtypeStruct((B, 128), jnp.float32),
        ],
    )(
        msc.reshape(B, 8, NV),
        rows.reshape(B, K, F),
        vals.reshape(B, K),
        idxs.reshape(B, K),
    )
    return out, pol[:, 0]
